# Initial kernel scaffold; baseline (speedup 1.0000x reference)
import jax
import jax.numpy as jnp
from jax.experimental import pallas as pl


def kernel(x, edge_index, W1, b1, gamma1, beta1, W2, b2, gamma2, beta2, prelu_a):
    raise NotImplementedError("write your pallas kernel here")


# speedup vs baseline: 21.0544x; 21.0544x over previous
import functools

import jax
import jax.numpy as jnp
from jax import lax
from jax.experimental import pallas as pl
from jax.experimental.pallas import tpu as pltpu
from jax.experimental.pallas import tpu_sc as plsc

N = 10000
E = 320000
D = 128
EPS = 1e-5

NC = 2
NS = 16
NW = NC * NS
PER_W = E // NW
K = 80
CH = PER_W // K
CHPB = 25
SCH = CH // CHPB
RPT = 624
RPT_LAST = N - RPT * (NS - 1)

_MESH = plsc.VectorSubcoreMesh(core_axis_name="c", subcore_axis_name="s")


def _tile_slab_copy(s, copy_fn):
    base = s * RPT

    @pl.when(s < NS - 1)
    def _():
        copy_fn(base, RPT)

    @pl.when(s == NS - 1)
    def _():
        copy_fn(base, RPT_LAST)


def _sc_degree_body(dst_hbm, zeros_hbm, ones_hbm, out_hbm, didx_v, ones_v,
                    acc):
    c = lax.axis_index("c")
    s = lax.axis_index("s")
    w = c * NS + s
    _tile_slab_copy(s, lambda base, size: pltpu.sync_copy(
        zeros_hbm.at[pl.ds(base, size)], acc.at[pl.ds(base, size)]))
    pltpu.sync_copy(ones_hbm, ones_v)
    pltpu.sync_copy(dst_hbm.at[w, 0], didx_v.at[0])
    plsc.subcore_barrier()

    def step(j, carry):
        t = lax.div(j, CHPB)
        r = lax.rem(j, CHPB)
        p = lax.rem(t, 2)
        pltpu.sync_copy(ones_v, acc.at[didx_v.at[p, r]], add=True)

        jn = j + 1
        tn = lax.div(jn, CHPB)
        rn = lax.rem(jn, CHPB)

        @pl.when((rn == 0) & (tn < SCH))
        def _():
            pltpu.sync_copy(dst_hbm.at[w, tn], didx_v.at[lax.rem(tn, 2)])

        return carry

    lax.fori_loop(0, CH, step, 0)
    plsc.subcore_barrier()
    _tile_slab_copy(s, lambda base, size: pltpu.sync_copy(
        acc.at[pl.ds(base, size)], out_hbm.at[c, pl.ds(base, size)]))


_sc_degree = functools.partial(
    pl.kernel,
    out_type=jax.ShapeDtypeStruct((NC, N, D), jnp.float32),
    mesh=_MESH,
    scratch_types=[
        pltpu.VMEM((2, CHPB, K), jnp.int32),
        pltpu.VMEM((K, D), jnp.float32),
        pltpu.VMEM_SHARED((N, D), jnp.float32),
    ],
)(_sc_degree_body)


def _sc_agg_body(g_hbm, src_hbm, dst_hbm, zeros_hbm, out_hbm, sidx_v, didx_v,
                 rows_v, gsem, isem, acc):
    c = lax.axis_index("c")
    s = lax.axis_index("s")
    w = c * NS + s
    _tile_slab_copy(s, lambda base, size: pltpu.sync_copy(
        zeros_hbm.at[pl.ds(base, size)], acc.at[pl.ds(base, size)]))
    pltpu.sync_copy(src_hbm.at[w, 0], sidx_v.at[0])
    pltpu.sync_copy(dst_hbm.at[w, 0], didx_v.at[0])
    plsc.subcore_barrier()

    def idx_start(t, p):
        pltpu.async_copy(src_hbm.at[w, t], sidx_v.at[p], isem)
        pltpu.async_copy(dst_hbm.at[w, t], didx_v.at[p], isem)

    def idx_wait(t, p):
        pltpu.make_async_copy(src_hbm.at[w, t], sidx_v.at[p], isem).wait()
        pltpu.make_async_copy(dst_hbm.at[w, t], didx_v.at[p], isem).wait()

    idx_start(1, 1)
    pltpu.async_copy(g_hbm.at[sidx_v.at[0, 0]], rows_v.at[0], gsem)

    def step(j, carry):
        cur = lax.rem(j, 2)
        t = lax.div(j, CHPB)
        r = lax.rem(j, CHPB)
        p = lax.rem(t, 2)
        pltpu.make_async_copy(g_hbm.at[sidx_v.at[p, r]], rows_v.at[cur],
                              gsem).wait()

        jn = j + 1
        tn = lax.div(jn, CHPB)
        rn = lax.rem(jn, CHPB)
        pn = lax.rem(tn, 2)

        @pl.when(jn < CH)
        def _():
            @pl.when(rn == 0)
            def _():
                idx_wait(tn, pn)

            pltpu.async_copy(g_hbm.at[sidx_v.at[pn, rn]], rows_v.at[1 - cur],
                             gsem)

        pltpu.sync_copy(rows_v.at[cur], acc.at[didx_v.at[p, r]], add=True)

        @pl.when((rn == 0) & (tn + 1 < SCH))
        def _():
            idx_start(tn + 1, lax.rem(tn + 1, 2))

        return carry

    lax.fori_loop(0, CH, step, 0)
    plsc.subcore_barrier()
    _tile_slab_copy(s, lambda base, size: pltpu.sync_copy(
        acc.at[pl.ds(base, size)], out_hbm.at[c, pl.ds(base, size)]))


_sc_agg = functools.partial(
    pl.kernel,
    out_type=jax.ShapeDtypeStruct((NC, N, D), jnp.float32),
    mesh=_MESH,
    scratch_types=[
        pltpu.VMEM((2, CHPB, K), jnp.int32),
        pltpu.VMEM((2, CHPB, K), jnp.int32),
        pltpu.VMEM((2, K, D), jnp.float32),
        pltpu.SemaphoreType.DMA,
        pltpu.SemaphoreType.DMA,
        pltpu.VMEM_SHARED((N, D), jnp.float32),
    ],
)(_sc_agg_body)


_BLK = 2000
_NBLK = N // _BLK


def _dinv_of(degp_blk):
    deg = degp_blk[0, :, 0] + degp_blk[1, :, 0] + 1.0
    return lax.rsqrt(deg)


def _tc_prep_body(x_ref, w1_ref, degp_ref, h_ref, g_ref):
    h = jnp.dot(x_ref[...], w1_ref[...].T, preferred_element_type=jnp.float32)
    dinv = _dinv_of(degp_ref[...])
    h_ref[...] = h
    g_ref[...] = h * dinv[:, None]


def _tc_prep(x, w1, degp):
    return pl.pallas_call(
        _tc_prep_body,
        grid=(_NBLK,),
        in_specs=[
            pl.BlockSpec((_BLK, D), lambda i: (i, 0)),
            pl.BlockSpec((D, D), lambda i: (0, 0)),
            pl.BlockSpec((NC, _BLK, D), lambda i: (0, i, 0)),
        ],
        out_specs=[
            pl.BlockSpec((_BLK, D), lambda i: (i, 0)),
            pl.BlockSpec((_BLK, D), lambda i: (i, 0)),
        ],
        out_shape=[
            jax.ShapeDtypeStruct((N, D), jnp.float32),
            jax.ShapeDtypeStruct((N, D), jnp.float32),
        ],
    )(x, w1, degp)


def _tc_stats_body(aggp_ref, h_ref, degp_ref, b_ref, z_ref, stats_ref,
                   acc_ref):
    i = pl.program_id(0)
    dinv = _dinv_of(degp_ref[...])
    agg = aggp_ref[0] + aggp_ref[1]
    z = agg * dinv[:, None] + h_ref[...] * (dinv * dinv)[:, None] + b_ref[...]
    z_ref[...] = z
    psum = jnp.sum(z, axis=0)
    psq = jnp.sum(z * z, axis=0)

    @pl.when(i == 0)
    def _():
        acc_ref[...] = jnp.zeros_like(acc_ref)

    acc_ref[0, :] += psum
    acc_ref[1, :] += psq
    stats_ref[...] = acc_ref[...]


def _tc_stats(aggp, h, degp, b):
    return pl.pallas_call(
        _tc_stats_body,
        grid=(_NBLK,),
        in_specs=[
            pl.BlockSpec((NC, _BLK, D), lambda i: (0, i, 0)),
            pl.BlockSpec((_BLK, D), lambda i: (i, 0)),
            pl.BlockSpec((NC, _BLK, D), lambda i: (0, i, 0)),
            pl.BlockSpec((1, D), lambda i: (0, 0)),
        ],
        out_specs=[
            pl.BlockSpec((_BLK, D), lambda i: (i, 0)),
            pl.BlockSpec((2, D), lambda i: (0, 0)),
        ],
        out_shape=[
            jax.ShapeDtypeStruct((N, D), jnp.float32),
            jax.ShapeDtypeStruct((2, D), jnp.float32),
        ],
        scratch_shapes=[pltpu.VMEM((2, D), jnp.float32)],
    )(aggp, h, degp, b.reshape(1, D))


def _bn_prelu(z, stats, gamma, beta, a):
    mean = stats[0, :] / N
    var = stats[1, :] / N - mean * mean
    y = (z - mean) * lax.rsqrt(var + EPS) * gamma + beta
    return jnp.where(y >= 0, y, a * y)


def _tc_apply_mm_body(z_ref, stats_ref, gamma_ref, beta_ref, a_ref, degp_ref,
                      w2_ref, h2_ref, g2_ref):
    y = _bn_prelu(z_ref[...], stats_ref[...], gamma_ref[0], beta_ref[0],
                  a_ref[0, 0])
    h2 = jnp.dot(y, w2_ref[...].T, preferred_element_type=jnp.float32)
    dinv = _dinv_of(degp_ref[...])
    h2_ref[...] = h2
    g2_ref[...] = h2 * dinv[:, None]


def _tc_apply_mm(z, stats, gamma, beta, a, degp, w2):
    return pl.pallas_call(
        _tc_apply_mm_body,
        grid=(_NBLK,),
        in_specs=[
            pl.BlockSpec((_BLK, D), lambda i: (i, 0)),
            pl.BlockSpec((2, D), lambda i: (0, 0)),
            pl.BlockSpec((1, D), lambda i: (0, 0)),
            pl.BlockSpec((1, D), lambda i: (0, 0)),
            pl.BlockSpec((1, 1), lambda i: (0, 0)),
            pl.BlockSpec((NC, _BLK, D), lambda i: (0, i, 0)),
            pl.BlockSpec((D, D), lambda i: (0, 0)),
        ],
        out_specs=[
            pl.BlockSpec((_BLK, D), lambda i: (i, 0)),
            pl.BlockSpec((_BLK, D), lambda i: (i, 0)),
        ],
        out_shape=[
            jax.ShapeDtypeStruct((N, D), jnp.float32),
            jax.ShapeDtypeStruct((N, D), jnp.float32),
        ],
    )(z, stats, gamma.reshape(1, D), beta.reshape(1, D), a.reshape(1, 1),
      degp, w2)


def _tc_apply_body(z_ref, stats_ref, gamma_ref, beta_ref, a_ref, y_ref):
    y_ref[...] = _bn_prelu(z_ref[...], stats_ref[...], gamma_ref[0],
                           beta_ref[0], a_ref[0, 0])


def _tc_apply(z, stats, gamma, beta, a):
    return pl.pallas_call(
        _tc_apply_body,
        grid=(_NBLK,),
        in_specs=[
            pl.BlockSpec((_BLK, D), lambda i: (i, 0)),
            pl.BlockSpec((2, D), lambda i: (0, 0)),
            pl.BlockSpec((1, D), lambda i: (0, 0)),
            pl.BlockSpec((1, D), lambda i: (0, 0)),
            pl.BlockSpec((1, 1), lambda i: (0, 0)),
        ],
        out_specs=pl.BlockSpec((_BLK, D), lambda i: (i, 0)),
        out_shape=jax.ShapeDtypeStruct((N, D), jnp.float32),
    )(z, stats, gamma.reshape(1, D), beta.reshape(1, D), a.reshape(1, 1))


@jax.jit
def _run(x, src, dst, W1, b1, gamma1, beta1, W2, b2, gamma2, beta2, prelu_a):
    src_r = src.reshape(NW, SCH, CHPB, K)
    dst_r = dst.reshape(NW, SCH, CHPB, K)
    zerosD = jnp.zeros((N, D), jnp.float32)
    onesD = jnp.ones((K, D), jnp.float32)

    degp = _sc_degree(dst_r, zerosD, onesD)
    h1, g1 = _tc_prep(x, W1, degp)
    agg1 = _sc_agg(g1, src_r, dst_r, zerosD)
    z1, stats1 = _tc_stats(agg1, h1, degp, b1)
    h2, g2 = _tc_apply_mm(z1, stats1, gamma1, beta1, prelu_a, degp, W2)
    agg2 = _sc_agg(g2, src_r, dst_r, zerosD)
    z2, stats2 = _tc_stats(agg2, h2, degp, b2)
    return _tc_apply(z2, stats2, gamma2, beta2, prelu_a)


def kernel(x, edge_index, W1, b1, gamma1, beta1, W2, b2, gamma2, beta2,
           prelu_a):
    src = edge_index[0].astype(jnp.int32)
    dst = edge_index[1].astype(jnp.int32)
    return _run(x, src, dst, W1, b1, gamma1, beta1, W2, b2, gamma2, beta2,
                prelu_a)


# speedup vs baseline: 28.0266x; 1.3312x over previous
import functools

import jax
import jax.numpy as jnp
from jax import lax
from jax.experimental import pallas as pl
from jax.experimental.pallas import tpu as pltpu
from jax.experimental.pallas import tpu_sc as plsc

N = 10000
E = 320000
D = 128
EPS = 1e-5

NC = 2
NS = 16
NW = NC * NS
PER_W = E // NW
K = 80
CH = PER_W // K
CHPB = 5
SCH = CH // CHPB
RPT = 624
RPT_LAST = N - RPT * (NS - 1)

_MESH = plsc.VectorSubcoreMesh(core_axis_name="c", subcore_axis_name="s")


def _tile_slab_copy(s, copy_fn):
    base = s * RPT

    @pl.when(s < NS - 1)
    def _():
        copy_fn(base, RPT)

    @pl.when(s == NS - 1)
    def _():
        copy_fn(base, RPT_LAST)


def _sc_degree_body(dst_hbm, zeros_hbm, ones_hbm, out_hbm, didx_v, ones_v,
                    ssem, acc):
    c = lax.axis_index("c")
    s = lax.axis_index("s")
    w = c * NS + s
    _tile_slab_copy(s, lambda base, size: pltpu.sync_copy(
        zeros_hbm.at[pl.ds(base, size)], acc.at[pl.ds(base, size)]))
    pltpu.sync_copy(ones_hbm, ones_v)
    pltpu.sync_copy(dst_hbm.at[w, 0], didx_v.at[0])
    plsc.subcore_barrier()

    def sadd(j):
        t = lax.div(j, CHPB)
        r = lax.rem(j, CHPB)
        p = lax.rem(t, 2)
        return pltpu.make_async_copy(ones_v, acc.at[didx_v.at[p, r]], ssem)

    def step(j, carry):
        pltpu.async_copy(ones_v,
                         acc.at[didx_v.at[lax.rem(lax.div(j, CHPB), 2),
                                          lax.rem(j, CHPB)]],
                         ssem, add=True)

        @pl.when(j >= 4)
        def _():
            sadd(j - 4).wait()

        jn = j + 1
        tn = lax.div(jn, CHPB)
        rn = lax.rem(jn, CHPB)

        @pl.when((rn == 0) & (tn < SCH))
        def _():
            pltpu.sync_copy(dst_hbm.at[w, tn], didx_v.at[lax.rem(tn, 2)])

        return carry

    lax.fori_loop(0, CH, step, 0)
    for dj in range(4):
        sadd(CH - 4 + dj).wait()
    plsc.subcore_barrier()
    _tile_slab_copy(s, lambda base, size: pltpu.sync_copy(
        acc.at[pl.ds(base, size)], out_hbm.at[c, pl.ds(base, size)]))


_sc_degree = functools.partial(
    pl.kernel,
    out_type=jax.ShapeDtypeStruct((NC, N, D), jnp.float32),
    mesh=_MESH,
    scratch_types=[
        pltpu.VMEM((2, CHPB, K), jnp.int32),
        pltpu.VMEM((K, D), jnp.float32),
        pltpu.SemaphoreType.DMA,
        pltpu.VMEM_SHARED((N, D), jnp.float32),
    ],
)(_sc_degree_body)


def _sc_agg_body(g_hbm, src_hbm, dst_hbm, zeros_hbm, out_hbm, sidx_v, didx_v,
                 rows_v, gsem, isem, ssem, acc):
    c = lax.axis_index("c")
    s = lax.axis_index("s")
    w = c * NS + s
    _tile_slab_copy(s, lambda base, size: pltpu.sync_copy(
        zeros_hbm.at[pl.ds(base, size)], acc.at[pl.ds(base, size)]))
    pltpu.sync_copy(src_hbm.at[w, 0], sidx_v.at[0])
    pltpu.sync_copy(dst_hbm.at[w, 0], didx_v.at[0])
    plsc.subcore_barrier()

    def idx_start(t, p):
        pltpu.async_copy(src_hbm.at[w, t], sidx_v.at[p], isem)
        pltpu.async_copy(dst_hbm.at[w, t], didx_v.at[p], isem)

    def idx_wait(t, p):
        pltpu.make_async_copy(src_hbm.at[w, t], sidx_v.at[p], isem).wait()
        pltpu.make_async_copy(dst_hbm.at[w, t], didx_v.at[p], isem).wait()

    def gref(j, buf):
        t = lax.div(j, CHPB)
        r = lax.rem(j, CHPB)
        return pltpu.make_async_copy(
            g_hbm.at[sidx_v.at[lax.rem(t, 3), r]], rows_v.at[buf], gsem)

    def sref(j, buf):
        t = lax.div(j, CHPB)
        r = lax.rem(j, CHPB)
        return pltpu.make_async_copy(
            rows_v.at[buf], acc.at[didx_v.at[lax.rem(t, 3), r]], ssem)

    idx_start(1, 1)
    pltpu.async_copy(g_hbm.at[sidx_v.at[0, 0]], rows_v.at[0], gsem)
    pltpu.async_copy(g_hbm.at[sidx_v.at[0, 1]], rows_v.at[1], gsem)

    def step(j, carry):
        a = lax.rem(j, 3)
        cc = lax.rem(j + 2, 3)
        gref(j, a).wait()
        pltpu.async_copy(rows_v.at[a],
                         acc.at[didx_v.at[lax.rem(lax.div(j, CHPB), 3),
                                          lax.rem(j, CHPB)]],
                         ssem, add=True)

        j2 = j + 2
        t2 = lax.div(j2, CHPB)
        r2 = lax.rem(j2, CHPB)

        @pl.when(j2 < CH)
        def _():
            @pl.when(j >= 1)
            def _():
                sref(j - 1, cc).wait()

            @pl.when(r2 == 0)
            def _():
                idx_wait(t2, lax.rem(t2, 3))

            pltpu.async_copy(g_hbm.at[sidx_v.at[lax.rem(t2, 3), r2]],
                             rows_v.at[cc], gsem)

            @pl.when((r2 == 0) & (t2 + 1 < SCH))
            def _():
                idx_start(t2 + 1, lax.rem(t2 + 1, 3))

        return carry

    lax.fori_loop(0, CH, step, 0)
    for dj in range(3):
        j = CH - 3 + dj
        sref(j, j % 3).wait()
    plsc.subcore_barrier()
    _tile_slab_copy(s, lambda base, size: pltpu.sync_copy(
        acc.at[pl.ds(base, size)], out_hbm.at[c, pl.ds(base, size)]))


_sc_agg = functools.partial(
    pl.kernel,
    out_type=jax.ShapeDtypeStruct((NC, N, D), jnp.float32),
    mesh=_MESH,
    scratch_types=[
        pltpu.VMEM((3, CHPB, K), jnp.int32),
        pltpu.VMEM((3, CHPB, K), jnp.int32),
        pltpu.VMEM((3, K, D), jnp.float32),
        pltpu.SemaphoreType.DMA,
        pltpu.SemaphoreType.DMA,
        pltpu.SemaphoreType.DMA,
        pltpu.VMEM_SHARED((N, D), jnp.float32),
    ],
)(_sc_agg_body)


_BLK = 2000
_NBLK = N // _BLK


def _dinv_of(degp_blk):
    deg = degp_blk[0, :, 0] + degp_blk[1, :, 0] + 1.0
    return lax.rsqrt(deg)


def _tc_prep_body(x_ref, w1_ref, degp_ref, h_ref, g_ref, dinv_ref):
    h = jnp.dot(x_ref[...], w1_ref[...].T, preferred_element_type=jnp.float32)
    dinv = _dinv_of(degp_ref[...])
    h_ref[...] = h
    g_ref[...] = h * dinv[:, None]
    dinv_ref[...] = dinv[:, None]


def _tc_prep(x, w1, degp):
    return pl.pallas_call(
        _tc_prep_body,
        grid=(_NBLK,),
        in_specs=[
            pl.BlockSpec((_BLK, D), lambda i: (i, 0)),
            pl.BlockSpec((D, D), lambda i: (0, 0)),
            pl.BlockSpec((NC, _BLK, D), lambda i: (0, i, 0)),
        ],
        out_specs=[
            pl.BlockSpec((_BLK, D), lambda i: (i, 0)),
            pl.BlockSpec((_BLK, D), lambda i: (i, 0)),
            pl.BlockSpec((_BLK, 1), lambda i: (i, 0)),
        ],
        out_shape=[
            jax.ShapeDtypeStruct((N, D), jnp.float32),
            jax.ShapeDtypeStruct((N, D), jnp.float32),
            jax.ShapeDtypeStruct((N, 1), jnp.float32),
        ],
    )(x, w1, degp)


def _tc_stats_body(aggp_ref, h_ref, dinv_ref, b_ref, z_ref, stats_ref,
                   acc_ref):
    i = pl.program_id(0)
    dinv = dinv_ref[...][:, 0]
    agg = aggp_ref[0] + aggp_ref[1]
    z = agg * dinv[:, None] + h_ref[...] * (dinv * dinv)[:, None] + b_ref[...]
    z_ref[...] = z
    psum = jnp.sum(z, axis=0)
    psq = jnp.sum(z * z, axis=0)

    @pl.when(i == 0)
    def _():
        acc_ref[...] = jnp.zeros_like(acc_ref)

    acc_ref[0, :] += psum
    acc_ref[1, :] += psq
    stats_ref[...] = acc_ref[...]


def _tc_stats(aggp, h, dinv, b):
    return pl.pallas_call(
        _tc_stats_body,
        grid=(_NBLK,),
        in_specs=[
            pl.BlockSpec((NC, _BLK, D), lambda i: (0, i, 0)),
            pl.BlockSpec((_BLK, D), lambda i: (i, 0)),
            pl.BlockSpec((_BLK, 1), lambda i: (i, 0)),
            pl.BlockSpec((1, D), lambda i: (0, 0)),
        ],
        out_specs=[
            pl.BlockSpec((_BLK, D), lambda i: (i, 0)),
            pl.BlockSpec((2, D), lambda i: (0, 0)),
        ],
        out_shape=[
            jax.ShapeDtypeStruct((N, D), jnp.float32),
            jax.ShapeDtypeStruct((2, D), jnp.float32),
        ],
        scratch_shapes=[pltpu.VMEM((2, D), jnp.float32)],
    )(aggp, h, dinv, b.reshape(1, D))


def _bn_prelu(z, stats, gamma, beta, a):
    mean = stats[0, :] / N
    var = stats[1, :] / N - mean * mean
    y = (z - mean) * lax.rsqrt(var + EPS) * gamma + beta
    return jnp.where(y >= 0, y, a * y)


def _tc_apply_mm_body(z_ref, stats_ref, gamma_ref, beta_ref, a_ref, dinv_ref,
                      w2_ref, h2_ref, g2_ref):
    y = _bn_prelu(z_ref[...], stats_ref[...], gamma_ref[0], beta_ref[0],
                  a_ref[0, 0])
    h2 = jnp.dot(y, w2_ref[...].T, preferred_element_type=jnp.float32)
    dinv = dinv_ref[...][:, 0]
    h2_ref[...] = h2
    g2_ref[...] = h2 * dinv[:, None]


def _tc_apply_mm(z, stats, gamma, beta, a, dinv, w2):
    return pl.pallas_call(
        _tc_apply_mm_body,
        grid=(_NBLK,),
        in_specs=[
            pl.BlockSpec((_BLK, D), lambda i: (i, 0)),
            pl.BlockSpec((2, D), lambda i: (0, 0)),
            pl.BlockSpec((1, D), lambda i: (0, 0)),
            pl.BlockSpec((1, D), lambda i: (0, 0)),
            pl.BlockSpec((1, 1), lambda i: (0, 0)),
            pl.BlockSpec((_BLK, 1), lambda i: (i, 0)),
            pl.BlockSpec((D, D), lambda i: (0, 0)),
        ],
        out_specs=[
            pl.BlockSpec((_BLK, D), lambda i: (i, 0)),
            pl.BlockSpec((_BLK, D), lambda i: (i, 0)),
        ],
        out_shape=[
            jax.ShapeDtypeStruct((N, D), jnp.float32),
            jax.ShapeDtypeStruct((N, D), jnp.float32),
        ],
    )(z, stats, gamma.reshape(1, D), beta.reshape(1, D), a.reshape(1, 1),
      dinv, w2)


def _tc_apply_body(z_ref, stats_ref, gamma_ref, beta_ref, a_ref, y_ref):
    y_ref[...] = _bn_prelu(z_ref[...], stats_ref[...], gamma_ref[0],
                           beta_ref[0], a_ref[0, 0])


def _tc_apply(z, stats, gamma, beta, a):
    return pl.pallas_call(
        _tc_apply_body,
        grid=(_NBLK,),
        in_specs=[
            pl.BlockSpec((_BLK, D), lambda i: (i, 0)),
            pl.BlockSpec((2, D), lambda i: (0, 0)),
            pl.BlockSpec((1, D), lambda i: (0, 0)),
            pl.BlockSpec((1, D), lambda i: (0, 0)),
            pl.BlockSpec((1, 1), lambda i: (0, 0)),
        ],
        out_specs=pl.BlockSpec((_BLK, D), lambda i: (i, 0)),
        out_shape=jax.ShapeDtypeStruct((N, D), jnp.float32),
    )(z, stats, gamma.reshape(1, D), beta.reshape(1, D), a.reshape(1, 1))


@jax.jit
def _run(x, src, dst, W1, b1, gamma1, beta1, W2, b2, gamma2, beta2, prelu_a):
    src_r = src.reshape(NW, SCH, CHPB, K)
    dst_r = dst.reshape(NW, SCH, CHPB, K)
    zerosD = jnp.zeros((N, D), jnp.float32)
    onesD = jnp.ones((K, D), jnp.float32)

    degp = _sc_degree(dst_r, zerosD, onesD)
    h1, g1, dinv = _tc_prep(x, W1, degp)
    agg1 = _sc_agg(g1, src_r, dst_r, zerosD)
    z1, stats1 = _tc_stats(agg1, h1, dinv, b1)
    h2, g2 = _tc_apply_mm(z1, stats1, gamma1, beta1, prelu_a, dinv, W2)
    agg2 = _sc_agg(g2, src_r, dst_r, zerosD)
    z2, stats2 = _tc_stats(agg2, h2, dinv, b2)
    return _tc_apply(z2, stats2, gamma2, beta2, prelu_a)


def kernel(x, edge_index, W1, b1, gamma1, beta1, W2, b2, gamma2, beta2,
           prelu_a):
    src = edge_index[0].astype(jnp.int32)
    dst = edge_index[1].astype(jnp.int32)
    return _run(x, src, dst, W1, b1, gamma1, beta1, W2, b2, gamma2, beta2,
                prelu_a)


# speedup vs baseline: 32.9793x; 1.1767x over previous
import functools

import jax
import jax.numpy as jnp
from jax import lax
from jax.experimental import pallas as pl
from jax.experimental.pallas import tpu as pltpu
from jax.experimental.pallas import tpu_sc as plsc

N = 10000
E = 320000
D = 128
EPS = 1e-5

NC = 2
NS = 16
NW = NC * NS
PER_W = E // NW
K = 80
CH = PER_W // K
CHPB = 5
SCH = CH // CHPB
RPT = 624
RPT_LAST = N - RPT * (NS - 1)

_MESH = plsc.VectorSubcoreMesh(core_axis_name="c", subcore_axis_name="s")


def _tile_slab_copy(s, copy_fn):
    base = s * RPT

    @pl.when(s < NS - 1)
    def _():
        copy_fn(base, RPT)

    @pl.when(s == NS - 1)
    def _():
        copy_fn(base, RPT_LAST)


NPAD = 80 * 128


def _sc_hist_body(dst_hbm, zeros_hbm, out_hbm, hist_v, idx_v):
    c = lax.axis_index("c")
    s = lax.axis_index("s")
    w = c * NS + s
    pltpu.sync_copy(zeros_hbm.at[pl.ds(0, 80)], hist_v)
    pltpu.sync_copy(dst_hbm.at[w], idx_v)
    ones = jnp.ones((16,), jnp.float32)

    def step(i, carry):
        r = lax.div(i, K // 16)
        q = lax.rem(i, K // 16)
        idx = idx_v[r, pl.ds(q * 16, 16)]
        row = lax.shift_right_logical(idx, 7)
        col = lax.bitwise_and(idx, 127)
        plsc.addupdate_scatter(hist_v, [row, col], ones)
        return carry

    lax.fori_loop(0, PER_W // 16, step, 0)
    pltpu.sync_copy(hist_v, out_hbm.at[w])


_sc_hist = functools.partial(
    pl.kernel,
    out_type=jax.ShapeDtypeStruct((NW, 80, 128), jnp.float32),
    mesh=_MESH,
    scratch_types=[
        pltpu.VMEM((80, 128), jnp.float32),
        pltpu.VMEM((CH, K), jnp.int32),
    ],
    compiler_params=pltpu.CompilerParams(needs_layout_passes=False),
)(_sc_hist_body)


def _sc_agg_body(g_hbm, src_hbm, dst_hbm, zeros_hbm, out_hbm, sidx_v, didx_v,
                 rows_v, gsem, isem, ssem, acc):
    c = lax.axis_index("c")
    s = lax.axis_index("s")
    w = c * NS + s
    _tile_slab_copy(s, lambda base, size: pltpu.sync_copy(
        zeros_hbm.at[pl.ds(base, size)], acc.at[pl.ds(base, size)]))
    pltpu.sync_copy(src_hbm.at[w, 0], sidx_v.at[0])
    pltpu.sync_copy(dst_hbm.at[w, 0], didx_v.at[0])
    plsc.subcore_barrier()

    def idx_start(t, p):
        pltpu.async_copy(src_hbm.at[w, t], sidx_v.at[p], isem)
        pltpu.async_copy(dst_hbm.at[w, t], didx_v.at[p], isem)

    def idx_wait(t, p):
        pltpu.make_async_copy(src_hbm.at[w, t], sidx_v.at[p], isem).wait()
        pltpu.make_async_copy(dst_hbm.at[w, t], didx_v.at[p], isem).wait()

    def gref(j, buf):
        t = lax.div(j, CHPB)
        r = lax.rem(j, CHPB)
        return pltpu.make_async_copy(
            g_hbm.at[sidx_v.at[lax.rem(t, 3), r]], rows_v.at[buf], gsem)

    def sref(j, buf):
        t = lax.div(j, CHPB)
        r = lax.rem(j, CHPB)
        return pltpu.make_async_copy(
            rows_v.at[buf], acc.at[didx_v.at[lax.rem(t, 3), r]], ssem)

    idx_start(1, 1)
    pltpu.async_copy(g_hbm.at[sidx_v.at[0, 0]], rows_v.at[0], gsem)
    pltpu.async_copy(g_hbm.at[sidx_v.at[0, 1]], rows_v.at[1], gsem)

    def step(j, carry):
        a = lax.rem(j, 3)
        cc = lax.rem(j + 2, 3)
        gref(j, a).wait()
        pltpu.async_copy(rows_v.at[a],
                         acc.at[didx_v.at[lax.rem(lax.div(j, CHPB), 3),
                                          lax.rem(j, CHPB)]],
                         ssem, add=True)

        j2 = j + 2
        t2 = lax.div(j2, CHPB)
        r2 = lax.rem(j2, CHPB)

        @pl.when(j2 < CH)
        def _():
            @pl.when(j >= 1)
            def _():
                sref(j - 1, cc).wait()

            @pl.when(r2 == 0)
            def _():
                idx_wait(t2, lax.rem(t2, 3))

            pltpu.async_copy(g_hbm.at[sidx_v.at[lax.rem(t2, 3), r2]],
                             rows_v.at[cc], gsem)

            @pl.when((r2 == 0) & (t2 + 1 < SCH))
            def _():
                idx_start(t2 + 1, lax.rem(t2 + 1, 3))

        return carry

    lax.fori_loop(0, CH, step, 0)
    for dj in range(3):
        j = CH - 3 + dj
        sref(j, j % 3).wait()
    plsc.subcore_barrier()
    _tile_slab_copy(s, lambda base, size: pltpu.sync_copy(
        acc.at[pl.ds(base, size)], out_hbm.at[c, pl.ds(base, size)]))


_sc_agg = functools.partial(
    pl.kernel,
    out_type=jax.ShapeDtypeStruct((NC, N, D), jnp.float32),
    mesh=_MESH,
    scratch_types=[
        pltpu.VMEM((3, CHPB, K), jnp.int32),
        pltpu.VMEM((3, CHPB, K), jnp.int32),
        pltpu.VMEM((3, K, D), jnp.float32),
        pltpu.SemaphoreType.DMA,
        pltpu.SemaphoreType.DMA,
        pltpu.SemaphoreType.DMA,
        pltpu.VMEM_SHARED((N, D), jnp.float32),
    ],
)(_sc_agg_body)


_BLK = 2000
_NBLK = N // _BLK


def _tc_dinv_body(histp_ref, dinv_ref):
    deg = jnp.sum(histp_ref[...], axis=0) + 1.0
    dinv_ref[...] = lax.rsqrt(deg)


def _tc_dinv(histp):
    return pl.pallas_call(
        _tc_dinv_body,
        out_shape=jax.ShapeDtypeStruct((80, 128), jnp.float32),
    )(histp)


def _tc_prep_body(x_ref, w1_ref, dinv_ref, h_ref, g_ref):
    h = jnp.dot(x_ref[...], w1_ref[...].T, preferred_element_type=jnp.float32)
    dinv = dinv_ref[...][:, 0]
    h_ref[...] = h
    g_ref[...] = h * dinv[:, None]


def _tc_prep(x, w1, dinv):
    return pl.pallas_call(
        _tc_prep_body,
        grid=(_NBLK,),
        in_specs=[
            pl.BlockSpec((_BLK, D), lambda i: (i, 0)),
            pl.BlockSpec((D, D), lambda i: (0, 0)),
            pl.BlockSpec((_BLK, 1), lambda i: (i, 0)),
        ],
        out_specs=[
            pl.BlockSpec((_BLK, D), lambda i: (i, 0)),
            pl.BlockSpec((_BLK, D), lambda i: (i, 0)),
        ],
        out_shape=[
            jax.ShapeDtypeStruct((N, D), jnp.float32),
            jax.ShapeDtypeStruct((N, D), jnp.float32),
        ],
    )(x, w1, dinv)


def _tc_stats_body(aggp_ref, h_ref, dinv_ref, b_ref, z_ref, stats_ref,
                   acc_ref):
    i = pl.program_id(0)
    dinv = dinv_ref[...][:, 0]
    agg = aggp_ref[0] + aggp_ref[1]
    z = agg * dinv[:, None] + h_ref[...] * (dinv * dinv)[:, None] + b_ref[...]
    z_ref[...] = z
    psum = jnp.sum(z, axis=0)
    psq = jnp.sum(z * z, axis=0)

    @pl.when(i == 0)
    def _():
        acc_ref[...] = jnp.zeros_like(acc_ref)

    acc_ref[0, :] += psum
    acc_ref[1, :] += psq
    stats_ref[...] = acc_ref[...]


def _tc_stats(aggp, h, dinv, b):
    return pl.pallas_call(
        _tc_stats_body,
        grid=(_NBLK,),
        in_specs=[
            pl.BlockSpec((NC, _BLK, D), lambda i: (0, i, 0)),
            pl.BlockSpec((_BLK, D), lambda i: (i, 0)),
            pl.BlockSpec((_BLK, 1), lambda i: (i, 0)),
            pl.BlockSpec((1, D), lambda i: (0, 0)),
        ],
        out_specs=[
            pl.BlockSpec((_BLK, D), lambda i: (i, 0)),
            pl.BlockSpec((2, D), lambda i: (0, 0)),
        ],
        out_shape=[
            jax.ShapeDtypeStruct((N, D), jnp.float32),
            jax.ShapeDtypeStruct((2, D), jnp.float32),
        ],
        scratch_shapes=[pltpu.VMEM((2, D), jnp.float32)],
    )(aggp, h, dinv, b.reshape(1, D))


def _bn_prelu(z, stats, gamma, beta, a):
    mean = stats[0, :] / N
    var = stats[1, :] / N - mean * mean
    y = (z - mean) * lax.rsqrt(var + EPS) * gamma + beta
    return jnp.where(y >= 0, y, a * y)


def _tc_apply_mm_body(z_ref, stats_ref, gamma_ref, beta_ref, a_ref, dinv_ref,
                      w2_ref, h2_ref, g2_ref):
    y = _bn_prelu(z_ref[...], stats_ref[...], gamma_ref[0], beta_ref[0],
                  a_ref[0, 0])
    h2 = jnp.dot(y, w2_ref[...].T, preferred_element_type=jnp.float32)
    dinv = dinv_ref[...][:, 0]
    h2_ref[...] = h2
    g2_ref[...] = h2 * dinv[:, None]


def _tc_apply_mm(z, stats, gamma, beta, a, dinv, w2):
    return pl.pallas_call(
        _tc_apply_mm_body,
        grid=(_NBLK,),
        in_specs=[
            pl.BlockSpec((_BLK, D), lambda i: (i, 0)),
            pl.BlockSpec((2, D), lambda i: (0, 0)),
            pl.BlockSpec((1, D), lambda i: (0, 0)),
            pl.BlockSpec((1, D), lambda i: (0, 0)),
            pl.BlockSpec((1, 1), lambda i: (0, 0)),
            pl.BlockSpec((_BLK, 1), lambda i: (i, 0)),
            pl.BlockSpec((D, D), lambda i: (0, 0)),
        ],
        out_specs=[
            pl.BlockSpec((_BLK, D), lambda i: (i, 0)),
            pl.BlockSpec((_BLK, D), lambda i: (i, 0)),
        ],
        out_shape=[
            jax.ShapeDtypeStruct((N, D), jnp.float32),
            jax.ShapeDtypeStruct((N, D), jnp.float32),
        ],
    )(z, stats, gamma.reshape(1, D), beta.reshape(1, D), a.reshape(1, 1),
      dinv, w2)


def _tc_apply_body(z_ref, stats_ref, gamma_ref, beta_ref, a_ref, y_ref):
    y_ref[...] = _bn_prelu(z_ref[...], stats_ref[...], gamma_ref[0],
                           beta_ref[0], a_ref[0, 0])


def _tc_apply(z, stats, gamma, beta, a):
    return pl.pallas_call(
        _tc_apply_body,
        grid=(_NBLK,),
        in_specs=[
            pl.BlockSpec((_BLK, D), lambda i: (i, 0)),
            pl.BlockSpec((2, D), lambda i: (0, 0)),
            pl.BlockSpec((1, D), lambda i: (0, 0)),
            pl.BlockSpec((1, D), lambda i: (0, 0)),
            pl.BlockSpec((1, 1), lambda i: (0, 0)),
        ],
        out_specs=pl.BlockSpec((_BLK, D), lambda i: (i, 0)),
        out_shape=jax.ShapeDtypeStruct((N, D), jnp.float32),
    )(z, stats, gamma.reshape(1, D), beta.reshape(1, D), a.reshape(1, 1))


@jax.jit
def _run(x, src, dst, W1, b1, gamma1, beta1, W2, b2, gamma2, beta2, prelu_a):
    src_r = src.reshape(NW, SCH, CHPB, K)
    dst_r = dst.reshape(NW, SCH, CHPB, K)
    dst_h = dst.reshape(NW, CH, K)
    zerosD = jnp.zeros((N, D), jnp.float32)

    histp = _sc_hist(dst_h, zerosD)
    dinv = _tc_dinv(histp).reshape(NPAD)[:N].reshape(N, 1)
    h1, g1 = _tc_prep(x, W1, dinv)
    agg1 = _sc_agg(g1, src_r, dst_r, zerosD)
    z1, stats1 = _tc_stats(agg1, h1, dinv, b1)
    h2, g2 = _tc_apply_mm(z1, stats1, gamma1, beta1, prelu_a, dinv, W2)
    agg2 = _sc_agg(g2, src_r, dst_r, zerosD)
    z2, stats2 = _tc_stats(agg2, h2, dinv, b2)
    return _tc_apply(z2, stats2, gamma2, beta2, prelu_a)


def kernel(x, edge_index, W1, b1, gamma1, beta1, W2, b2, gamma2, beta2,
           prelu_a):
    src = edge_index[0].astype(jnp.int32)
    dst = edge_index[1].astype(jnp.int32)
    return _run(x, src, dst, W1, b1, gamma1, beta1, W2, b2, gamma2, beta2,
                prelu_a)
